# baseline (speedup 1.0000x reference)
import functools

import jax
import jax.numpy as jnp
from jax import lax
from jax.experimental import pallas as pl
from jax.experimental.pallas import tpu as pltpu
from jax.experimental.pallas import tpu_sc as plsc

N_NODES = 100000
D = 128
NSEG = 256
NC = 2
NS = 16
L = 16

COLS = D // NC
CHUNK = 512
NBUF = 2
ROWS_MAIN = 6256
ROWS_LAST = N_NODES - (NS - 1) * ROWS_MAIN
N_CHUNKS = -(-ROWS_MAIN // CHUNK)


@jax.jit
def _sum_nodes_sc(feat, sids):
    mesh = plsc.VectorSubcoreMesh(core_axis_name="c", subcore_axis_name="s")

    @functools.partial(
        pl.kernel,
        out_type=jax.ShapeDtypeStruct((NSEG, D), jnp.float32),
        mesh=mesh,
        compiler_params=pltpu.CompilerParams(
            use_tc_tiling_on_sc=False, needs_layout_passes=False),
        scratch_types=[
            pltpu.VMEM((NSEG, COLS), jnp.float32),
            pltpu.VMEM((NBUF, CHUNK, COLS), jnp.float32),
            pltpu.VMEM((NBUF, CHUNK), jnp.int32),
            pltpu.VMEM((L, COLS), jnp.float32),
            pltpu.VMEM((L, COLS), jnp.float32),
            pltpu.VMEM_SHARED((NS, NSEG, COLS), jnp.float32),
            pltpu.SemaphoreType.DMA((NBUF,)),
            pltpu.SemaphoreType.DMA((NBUF,)),
        ],
    )
    def k(feat_hbm, sid_hbm, out_hbm, acc, bufs, sbufs, tmp, racc, shared,
          semf, semi):
        c = lax.axis_index("c")
        s = lax.axis_index("s")
        col0 = c * COLS
        start = s * ROWS_MAIN
        rows = jnp.where(s == NS - 1, ROWS_LAST, ROWS_MAIN)

        def issue(kc, b):
            off = jnp.minimum(kc * CHUNK, rows - CHUNK)
            row0 = start + off
            pltpu.async_copy(
                feat_hbm.at[pl.ds(row0, CHUNK), pl.ds(col0, COLS)],
                bufs.at[b], semf.at[b])
            pltpu.async_copy(sid_hbm.at[pl.ds(row0, CHUNK)],
                             sbufs.at[b], semi.at[b])

        for n in range(NBUF - 1):
            issue(n, n)

        z = jnp.zeros((L,), jnp.float32)

        def zero_row(i, carry):
            for j in range(COLS // L):
                acc[i, pl.ds(j * L, L)] = z
            return carry

        lax.fori_loop(0, NSEG, zero_row, 0)

        col_idx = [j * L + lax.iota(jnp.int32, L) for j in range(COLS // L)]
        bcast_idx = [jnp.full((L, 1), i, dtype=jnp.int32) for i in range(L)]
        dnums = lax.GatherDimensionNumbers(
            offset_dims=(), collapsed_slice_dims=(0,), start_index_map=(0,))

        def bcast_lane(v, i):
            return lax.gather(
                v, bcast_idx[i], dnums, slice_sizes=(1,),
                mode=lax.GatherScatterMode.PROMISE_IN_BOUNDS)

        def chunk_body(kc, carry):
            b = lax.rem(kc, NBUF)
            pltpu.make_async_copy(
                feat_hbm.at[pl.ds(0, CHUNK), pl.ds(col0, COLS)],
                bufs.at[b], semf.at[b]).wait()
            pltpu.make_async_copy(
                sid_hbm.at[pl.ds(0, CHUNK)], sbufs.at[b], semi.at[b]).wait()

            nxt = kc + NBUF - 1

            @pl.when(nxt < N_CHUNKS)
            def _():
                issue(nxt, lax.rem(nxt, NBUF))

            g0 = jnp.maximum(0, (kc + 1) * CHUNK - rows) // L

            @plsc.parallel_loop(g0, CHUNK // L)
            def group_body(g):
                r0 = g * L
                segv = sbufs[b, pl.ds(r0, L)]
                segbs = [bcast_lane(segv, i) for i in range(L)]
                prev = None
                for j in range(COLS // L + 1):
                    vals = None
                    if j < COLS // L:
                        vals = [bufs[b, r0 + i, pl.ds(j * L, L)]
                                for i in range(L)]
                    if prev is not None:
                        for i in range(L):
                            plsc.addupdate_scatter(
                                acc, [segbs[i], col_idx[j - 1]], prev[i])
                    prev = vals

            return carry

        lax.fori_loop(0, N_CHUNKS, chunk_body, 0)

        pltpu.sync_copy(acc, shared.at[s])
        plsc.subcore_barrier()

        pltpu.sync_copy(shared.at[0, pl.ds(s * L, L), :], racc)

        def red_body(kk, carry):
            pltpu.sync_copy(shared.at[kk, pl.ds(s * L, L), :], tmp)
            for i in range(L):
                for j in range(COLS // L):
                    plsc.addupdate(
                        racc.at[i, pl.ds(j * L, L)],
                        tmp[i, pl.ds(j * L, L)])
            return carry

        lax.fori_loop(1, NS, red_body, 0)

        pltpu.sync_copy(
            racc, out_hbm.at[pl.ds(s * L, L), pl.ds(col0, COLS)])

    return k(feat, sids)


def kernel(feat, segment_ids):
    return _sum_nodes_sc(feat, segment_ids.astype(jnp.int32))


# speedup vs baseline: 1.1302x; 1.1302x over previous
import functools

import jax
import jax.numpy as jnp
from jax import lax
from jax.experimental import pallas as pl
from jax.experimental.pallas import tpu as pltpu
from jax.experimental.pallas import tpu_sc as plsc

N_NODES = 100000
D = 128
NSEG = 256
NC = 2
NS = 16
L = 16

COLS = D // NC
CHUNK = 512
NBUF = 2
ROWS_MAIN = 6256
ROWS_LAST = N_NODES - (NS - 1) * ROWS_MAIN
N_CHUNKS = -(-ROWS_MAIN // CHUNK)


@jax.jit
def _sum_nodes_sc(feat, sids):
    mesh = plsc.VectorSubcoreMesh(core_axis_name="c", subcore_axis_name="s")

    @functools.partial(
        pl.kernel,
        out_type=jax.ShapeDtypeStruct((NSEG, D), jnp.float32),
        mesh=mesh,
        compiler_params=pltpu.CompilerParams(
            use_tc_tiling_on_sc=False, needs_layout_passes=False),
        scratch_types=[
            pltpu.VMEM((NSEG, COLS), jnp.float32),
            pltpu.VMEM((NBUF, CHUNK, COLS), jnp.float32),
            pltpu.VMEM((NBUF, CHUNK), jnp.int32),
            pltpu.VMEM((L, COLS), jnp.float32),
            pltpu.VMEM((L, COLS), jnp.float32),
            pltpu.VMEM_SHARED((NS, NSEG, COLS), jnp.float32),
            pltpu.SemaphoreType.DMA((NBUF,)),
            pltpu.SemaphoreType.DMA((NBUF,)),
        ],
    )
    def k(feat_hbm, sid_hbm, out_hbm, acc, bufs, sbufs, tmp, racc, shared,
          semf, semi):
        c = lax.axis_index("c")
        s = lax.axis_index("s")
        col0 = c * COLS
        start = s * ROWS_MAIN
        rows = jnp.where(s == NS - 1, ROWS_LAST, ROWS_MAIN)

        def issue(kc, b):
            off = jnp.minimum(kc * CHUNK, rows - CHUNK)
            row0 = start + off
            pltpu.async_copy(
                feat_hbm.at[pl.ds(row0, CHUNK), pl.ds(col0, COLS)],
                bufs.at[b], semf.at[b])
            pltpu.async_copy(sid_hbm.at[pl.ds(row0, CHUNK)],
                             sbufs.at[b], semi.at[b])

        for n in range(NBUF - 1):
            issue(n, n)

        z = jnp.zeros((L,), jnp.float32)

        def zero_row(i, carry):
            for j in range(COLS // L):
                acc[i, pl.ds(j * L, L)] = z
            return carry

        lax.fori_loop(0, NSEG, zero_row, 0)

        col_idx = [j * L + lax.iota(jnp.int32, L) for j in range(COLS // L)]
        bcast_idx = [jnp.full((L, 1), i, dtype=jnp.int32) for i in range(L)]
        dnums = lax.GatherDimensionNumbers(
            offset_dims=(), collapsed_slice_dims=(0,), start_index_map=(0,))

        def bcast_lane(v, i):
            return lax.gather(
                v, bcast_idx[i], dnums, slice_sizes=(1,),
                mode=lax.GatherScatterMode.PROMISE_IN_BOUNDS)

        def chunk_body(kc, carry):
            b = lax.rem(kc, NBUF)
            pltpu.make_async_copy(
                feat_hbm.at[pl.ds(0, CHUNK), pl.ds(col0, COLS)],
                bufs.at[b], semf.at[b]).wait()
            pltpu.make_async_copy(
                sid_hbm.at[pl.ds(0, CHUNK)], sbufs.at[b], semi.at[b]).wait()

            nxt = kc + NBUF - 1

            @pl.when(nxt < N_CHUNKS)
            def _():
                issue(nxt, lax.rem(nxt, NBUF))

            g0 = jnp.maximum(0, (kc + 1) * CHUNK - rows) // L

            @plsc.parallel_loop(g0, CHUNK // L)
            def group_body(g):
                r0 = g * L
                segv = sbufs[b, pl.ds(r0, L)]
                segbs = [bcast_lane(segv, i) for i in range(L)]
                for j in range(COLS // L):
                    for i in range(L):
                        plsc.addupdate_scatter(
                            acc, [segbs[i], col_idx[j]],
                            bufs[b, r0 + i, pl.ds(j * L, L)])

            return carry

        lax.fori_loop(0, N_CHUNKS, chunk_body, 0)

        pltpu.sync_copy(acc, shared.at[s])
        plsc.subcore_barrier()

        pltpu.sync_copy(shared.at[0, pl.ds(s * L, L), :], racc)

        def red_body(kk, carry):
            pltpu.sync_copy(shared.at[kk, pl.ds(s * L, L), :], tmp)
            for i in range(L):
                for j in range(COLS // L):
                    plsc.addupdate(
                        racc.at[i, pl.ds(j * L, L)],
                        tmp[i, pl.ds(j * L, L)])
            return carry

        lax.fori_loop(1, NS, red_body, 0)

        pltpu.sync_copy(
            racc, out_hbm.at[pl.ds(s * L, L), pl.ds(col0, COLS)])

    return k(feat, sids)


def kernel(feat, segment_ids):
    return _sum_nodes_sc(feat, segment_ids.astype(jnp.int32))
